# baseline (speedup 1.0000x reference)
import functools

import jax
import jax.numpy as jnp
from jax import lax
from jax.experimental import pallas as pl
from jax.experimental.pallas import tpu as pltpu
from jax.experimental.pallas import tpu_sc as plsc

B, L, D = 4096, 200, 32
NC, NS = 2, 16
NW = NC * NS
SEQ_W = B // NW
RT, IT = D // 8, 8
CT, JT = B // 128, 128
NPAIR = L // 2
G_BYTES = SEQ_W * D * 4

_mesh = plsc.VectorSubcoreMesh(core_axis_name="c", subcore_axis_name="s")


@functools.partial(
    pl.kernel,
    mesh=_mesh,
    out_type=jax.ShapeDtypeStruct((L, RT, CT, IT, JT), jnp.float32),
    scratch_types=[
        pltpu.VMEM((L, SEQ_W), jnp.int32),
        pltpu.VMEM((SEQ_W, D), jnp.float32),
        pltpu.VMEM((SEQ_W, D), jnp.float32),
        pltpu.VMEM((RT, IT, JT), jnp.float32),
        pltpu.VMEM((RT, IT, JT), jnp.float32),
        pltpu.HBM((SEQ_W, D), jnp.float32),
        pltpu.SemaphoreType.DMA,
        pltpu.SemaphoreType.DMA,
        pltpu.SemaphoreType.DMA,
        pltpu.SemaphoreType.DMA,
    ],
    compiler_params=pltpu.CompilerParams(
        use_tc_tiling_on_sc=False, needs_layout_passes=False),
)
def _gather_kernel(table_hbm, idx_hbm, out_hbm, idx_v, g0, g1, t0, t1,
                   dummy_hbm, sem_g0, sem_g1, sem_w0, sem_w1):
    wid = lax.axis_index("s") * NC + lax.axis_index("c")
    pltpu.sync_copy(idx_hbm.at[pl.ds(0, L), pl.ds(wid * SEQ_W, SEQ_W)], idx_v)

    jvecs = [lax.iota(jnp.int32, 16) + j0 for j0 in range(0, SEQ_W, 16)]

    def transpose(g, t):
        for dd in range(D):
            ddvec = jnp.full((16,), dd, jnp.int32)
            for k, jvec in enumerate(jvecs):
                t[dd // IT, dd % IT, pl.ds(k * 16, 16)] = plsc.load_gather(
                    g, [jvec, ddvec])

    def fire(l, g, sem):
        return pltpu.async_copy(table_hbm.at[idx_v.at[l]], g, sem)

    def drain_g(g, sem):
        pltpu.make_async_copy(dummy_hbm, g, sem).wait()

    def drain_w(t, sem):
        pltpu.make_async_copy(out_hbm.at[0, pl.ds(0, RT), 0], t, sem).wait()

    fire(0, g0, sem_g0)

    def body(i, carry):
        l0 = 2 * i
        fire(l0 + 1, g1, sem_g1)

        drain_g(g0, sem_g0)

        @pl.when(i > 0)
        def _():
            drain_w(t0, sem_w0)

        transpose(g0, t0)
        pltpu.async_copy(t0, out_hbm.at[l0, pl.ds(0, RT), wid], sem_w0)

        @pl.when(i < NPAIR - 1)
        def _():
            fire(l0 + 2, g0, sem_g0)

        drain_g(g1, sem_g1)

        @pl.when(i > 0)
        def _():
            drain_w(t1, sem_w1)

        transpose(g1, t1)
        pltpu.async_copy(t1, out_hbm.at[l0 + 1, pl.ds(0, RT), wid], sem_w1)
        return carry

    lax.fori_loop(0, NPAIR, body, 0)
    drain_w(t0, sem_w0)
    drain_w(t1, sem_w1)


def kernel(input_seqs, table):
    out5 = _gather_kernel(table, input_seqs.T)
    return out5.transpose(2, 4, 0, 1, 3).reshape(B, L, D)


# speedup vs baseline: 1.0391x; 1.0391x over previous
import functools

import jax
import jax.numpy as jnp
from jax import lax
from jax.experimental import pallas as pl
from jax.experimental.pallas import tpu as pltpu
from jax.experimental.pallas import tpu_sc as plsc

B, L, D = 4096, 200, 32
NC, NS = 2, 16
NW = NC * NS
SEQ_W = B // NW
RT, IT = D // 8, 8
CT, JT = B // 128, 128
LT = L // 8
NPAIR = (LT - 1) // 2

_mesh = plsc.VectorSubcoreMesh(core_axis_name="c", subcore_axis_name="s")


@functools.partial(
    pl.kernel,
    mesh=_mesh,
    out_type=jax.ShapeDtypeStruct((L, RT, CT, IT, JT), jnp.float32),
    scratch_types=[
        pltpu.VMEM((IT, JT), jnp.int32),
        pltpu.VMEM((IT, JT), jnp.int32),
        pltpu.VMEM((IT * JT, D), jnp.float32),
        pltpu.VMEM((IT * JT, D), jnp.float32),
        pltpu.VMEM((IT, RT, IT, JT), jnp.float32),
        pltpu.HBM((IT * JT, D), jnp.float32),
        pltpu.SemaphoreType.DMA,
        pltpu.SemaphoreType.DMA,
        pltpu.SemaphoreType.DMA,
        pltpu.SemaphoreType.DMA,
        pltpu.SemaphoreType.DMA,
    ],
    compiler_params=pltpu.CompilerParams(
        use_tc_tiling_on_sc=False, needs_layout_passes=False),
)
def _gather_kernel(table_hbm, idx_hbm, out_hbm, ix0, ix1, g0, g1, tb,
                   dummy_hbm, sem_i0, sem_i1, sem_g0, sem_g1, sem_w):
    wid = lax.axis_index("s") * NC + lax.axis_index("c")

    jvecs = [lax.iota(jnp.int32, 16) + j0 for j0 in range(0, JT, 16)]

    def fire_idx(grp, ix, sem):
        return pltpu.async_copy(idx_hbm.at[grp, wid], ix, sem)

    def fire_gathers(ix, g, sem):
        for i in range(IT):
            pltpu.async_copy(
                table_hbm.at[ix.at[i]], g.at[pl.ds(i * JT, JT)], sem)

    def transpose_block(c, g):
        cbase = jnp.full((16,), 0, jnp.int32) + c * JT
        jv_c = [jvec + cbase for jvec in jvecs]
        for dd in range(D):
            ddvec = jnp.full((16,), dd, jnp.int32)
            for k, jvec in enumerate(jv_c):
                tb[c, dd // IT, dd % IT, pl.ds(k * 16, 16)] = plsc.load_gather(
                    g, [jvec, ddvec])

    def transpose_group(g):
        def tbody(c, carry):
            transpose_block(c, g)
            return carry
        lax.fori_loop(0, IT, tbody, 0)

    def fire_write(grp, sem):
        return pltpu.async_copy(
            tb, out_hbm.at[pl.ds(grp * IT, IT), pl.ds(0, RT), wid], sem)

    def drain_i(ix, sem):
        pltpu.make_async_copy(idx_hbm.at[0, 0], ix, sem).wait()

    def drain_g(g, sem):
        pltpu.make_async_copy(dummy_hbm, g, sem).wait()

    def drain_w():
        pltpu.make_async_copy(
            out_hbm.at[pl.ds(0, IT), pl.ds(0, RT), 0], tb, sem_w).wait()

    fire_idx(0, ix0, sem_i0).wait()
    fire_gathers(ix0, g0, sem_g0)
    fire_idx(1, ix1, sem_i1)

    def body(i, carry):
        a = 2 * i
        drain_g(g0, sem_g0)
        fire_idx(a + 2, ix0, sem_i0)
        drain_i(ix1, sem_i1)
        fire_gathers(ix1, g1, sem_g1)

        @pl.when(i > 0)
        def _():
            drain_w()

        transpose_group(g0)
        fire_write(a, sem_w)

        drain_g(g1, sem_g1)
        drain_i(ix0, sem_i0)
        fire_gathers(ix0, g0, sem_g0)

        @pl.when(a + 3 < LT)
        def _():
            fire_idx(a + 3, ix1, sem_i1)

        drain_w()
        transpose_group(g1)
        fire_write(a + 1, sem_w)
        return carry

    lax.fori_loop(0, NPAIR, body, 0)

    drain_g(g0, sem_g0)
    drain_w()
    transpose_group(g0)
    fire_write(LT - 1, sem_w)
    drain_w()


def kernel(input_seqs, table):
    idx4 = input_seqs.T.reshape(LT, IT, CT, JT).transpose(0, 2, 1, 3)
    out5 = _gather_kernel(table, idx4)
    return out5.transpose(2, 4, 0, 1, 3).reshape(B, L, D)


# speedup vs baseline: 1.1169x; 1.0748x over previous
import functools

import jax
import jax.numpy as jnp
from jax import lax
from jax.experimental import pallas as pl
from jax.experimental.pallas import tpu as pltpu
from jax.experimental.pallas import tpu_sc as plsc

B, L, D = 4096, 200, 32
NC, NS = 2, 16
NW = NC * NS
SEQ_W = B // NW
RT, IT = D // 8, 8
CT, JT = B // 128, 128
LT = L // 8
NPAIR = (LT - 1) // 2
TT = 5

_mesh = plsc.VectorSubcoreMesh(core_axis_name="c", subcore_axis_name="s")


@functools.partial(
    pl.kernel,
    mesh=_mesh,
    out_type=jax.ShapeDtypeStruct((LT, CT, IT, JT, D), jnp.float32),
    scratch_types=[
        pltpu.VMEM((IT, JT), jnp.int32),
        pltpu.VMEM((IT, JT), jnp.int32),
        pltpu.VMEM((IT, JT, D), jnp.float32),
        pltpu.VMEM((IT, JT, D), jnp.float32),
        pltpu.HBM((IT, JT, D), jnp.float32),
        pltpu.SemaphoreType.DMA,
        pltpu.SemaphoreType.DMA,
        pltpu.SemaphoreType.DMA,
        pltpu.SemaphoreType.DMA,
        pltpu.SemaphoreType.DMA,
        pltpu.SemaphoreType.DMA,
    ],
    compiler_params=pltpu.CompilerParams(
        use_tc_tiling_on_sc=False, needs_layout_passes=False),
)
def _gather_kernel(table_hbm, idx_hbm, out_hbm, ix0, ix1, g0, g1,
                   dummy_hbm, sem_i0, sem_i1, sem_g0, sem_g1,
                   sem_w0, sem_w1):
    wid = lax.axis_index("s") * NC + lax.axis_index("c")

    def fire_idx(grp, ix, sem):
        return pltpu.async_copy(idx_hbm.at[grp, wid], ix, sem)

    def fire_gathers(ix, g, sem):
        for i in range(IT):
            pltpu.async_copy(table_hbm.at[ix.at[i]], g.at[i], sem)

    def fire_write(grp, g, sem):
        return pltpu.async_copy(g, out_hbm.at[grp, wid], sem)

    def drain_i(ix, sem):
        pltpu.make_async_copy(idx_hbm.at[0, 0], ix, sem).wait()

    def drain_g(g, sem):
        pltpu.make_async_copy(dummy_hbm, g, sem).wait()

    def drain_w(g, sem):
        pltpu.make_async_copy(dummy_hbm, g, sem).wait()

    fire_idx(0, ix0, sem_i0)
    drain_i(ix0, sem_i0)
    fire_gathers(ix0, g0, sem_g0)
    fire_idx(1, ix1, sem_i1)

    def body(i, carry):
        a = 2 * i
        drain_i(ix1, sem_i1)

        @pl.when(i > 0)
        def _():
            drain_w(g1, sem_w1)

        fire_gathers(ix1, g1, sem_g1)
        drain_g(g0, sem_g0)
        fire_idx(a + 2, ix0, sem_i0)
        fire_write(a, g0, sem_w0)

        drain_i(ix0, sem_i0)
        drain_w(g0, sem_w0)
        fire_gathers(ix0, g0, sem_g0)
        drain_g(g1, sem_g1)

        @pl.when(a + 3 < LT)
        def _():
            fire_idx(a + 3, ix1, sem_i1)

        fire_write(a + 1, g1, sem_w1)
        return carry

    lax.fori_loop(0, NPAIR, body, 0)

    drain_g(g0, sem_g0)
    drain_w(g1, sem_w1)
    fire_write(LT - 1, g0, sem_w0)
    drain_w(g0, sem_w0)


def _relayout_body(x_ref, o_ref):
    x = x_ref[...].reshape(TT * IT, JT, D)
    o_ref[...] = jnp.swapaxes(x, 1, 2).reshape(TT * IT, RT, 1, IT, JT)


_relayout = pl.pallas_call(
    _relayout_body,
    grid=(CT, LT // TT),
    in_specs=[pl.BlockSpec(
        (TT, 1, IT, JT, D), lambda c, t: (t, c, 0, 0, 0))],
    out_specs=pl.BlockSpec(
        (TT * IT, RT, 1, IT, JT), lambda c, t: (t, 0, c, 0, 0)),
    out_shape=jax.ShapeDtypeStruct((L, RT, CT, IT, JT), jnp.float32),
)


def kernel(input_seqs, table):
    idx4 = input_seqs.T.reshape(LT, IT, CT, JT).transpose(0, 2, 1, 3)
    staged = _gather_kernel(table, idx4)
    out5 = _relayout(staged)
    return out5.transpose(2, 4, 0, 1, 3).reshape(B, L, D)
